# baseline (speedup 1.0000x reference)
import functools

import jax
import jax.numpy as jnp
from jax import lax
from jax.experimental import pallas as pl
from jax.experimental.pallas import tpu as pltpu
from jax.experimental.pallas import tpu_sc as plsc

D_MODEL = 1024
NC = 2
NS = 16
NW = NC * NS
K = 16
N_CHUNKS = 64
NBUF = 4
G = 1
W = 3


def _pe_gather(idx3, table):
    B = NW * N_CHUNKS * K
    mesh = plsc.VectorSubcoreMesh(core_axis_name="c", subcore_axis_name="s")

    @functools.partial(
        pl.kernel,
        mesh=mesh,
        out_type=jax.ShapeDtypeStruct((B, D_MODEL), jnp.float32),
        scratch_types=(
            [pltpu.VMEM((N_CHUNKS, K), jnp.int32)]
            + [pltpu.VMEM((K, D_MODEL), jnp.float32) for _ in range(NBUF)]
            + [pltpu.SemaphoreType.DMA for _ in range(2 * NBUF)]
        ),
    )
    def body(idx_hbm, table_hbm, out_hbm, idx_v, *rest):
        bufs = rest[:NBUF]
        gsems = rest[NBUF:2 * NBUF]
        wsems = rest[2 * NBUF:]
        wid = lax.axis_index("s") * NC + lax.axis_index("c")
        base = wid * (N_CHUNKS * K)
        pltpu.sync_copy(idx_hbm.at[wid], idx_v)

        def gather(c, b):
            return pltpu.async_copy(table_hbm.at[idx_v.at[c]], bufs[b], gsems[b])

        def wait_gather(c, b):
            pltpu.make_async_copy(
                table_hbm.at[idx_v.at[c]], bufs[b], gsems[b]).wait()

        def write(c, b):
            return pltpu.async_copy(
                bufs[b], out_hbm.at[pl.ds(base + c * K, K)], wsems[b])

        def wait_write(c, b):
            pltpu.make_async_copy(
                bufs[b], out_hbm.at[pl.ds(base + c * K, K)], wsems[b]).wait()

        for c in range(G):
            gather(c, c % NBUF)

        def step(c, j):
            wait_gather(c, j % NBUF)
            write(c, j % NBUF)

            @pl.when(c >= W)
            def _():
                wait_write(c - W, (j - W) % NBUF)

            @pl.when(c + G < N_CHUNKS)
            def _():
                gather(c + G, (j + G) % NBUF)

        def ring(i, carry):
            for j in range(NBUF):
                step(NBUF * i + j, j)
            return carry

        n_rounds = N_CHUNKS // NBUF
        lax.fori_loop(0, n_rounds, ring, 0)
        for c in range(n_rounds * NBUF, N_CHUNKS):
            step(jnp.int32(c), c % NBUF)
        for c in range(N_CHUNKS - W, N_CHUNKS):
            wait_write(jnp.int32(c), c % NBUF)

    return body(idx3, table)


def kernel(token_positions, PE):
    idx3 = token_positions.reshape(NW, N_CHUNKS, K)
    out = _pe_gather(idx3, PE)
    return out.reshape(token_positions.shape + (D_MODEL,))


# speedup vs baseline: 1.2347x; 1.2347x over previous
import functools

import jax
import jax.numpy as jnp
from jax import lax
from jax.experimental import pallas as pl
from jax.experimental.pallas import tpu as pltpu
from jax.experimental.pallas import tpu_sc as plsc

D_MODEL = 1024
NC = 2
NS = 16
NW = NC * NS
K = 16
N_CHUNKS = 64
NBUF = 4
G = 3
W = 1


def _pe_gather(idx3, table):
    B = NW * N_CHUNKS * K
    mesh = plsc.VectorSubcoreMesh(core_axis_name="c", subcore_axis_name="s")

    @functools.partial(
        pl.kernel,
        mesh=mesh,
        out_type=jax.ShapeDtypeStruct((B, D_MODEL), jnp.float32),
        scratch_types=(
            [pltpu.VMEM((N_CHUNKS, K), jnp.int32)]
            + [pltpu.VMEM((K, D_MODEL), jnp.float32) for _ in range(NBUF)]
            + [pltpu.SemaphoreType.DMA for _ in range(2 * NBUF)]
        ),
    )
    def body(idx_hbm, table_hbm, out_hbm, idx_v, *rest):
        bufs = rest[:NBUF]
        gsems = rest[NBUF:2 * NBUF]
        wsems = rest[2 * NBUF:]
        wid = lax.axis_index("s") * NC + lax.axis_index("c")
        base = wid * (N_CHUNKS * K)
        pltpu.sync_copy(idx_hbm.at[wid], idx_v)

        def gather(c, b):
            return pltpu.async_copy(table_hbm.at[idx_v.at[c]], bufs[b], gsems[b])

        def wait_gather(c, b):
            pltpu.make_async_copy(
                table_hbm.at[idx_v.at[c]], bufs[b], gsems[b]).wait()

        def write(c, b):
            return pltpu.async_copy(
                bufs[b], out_hbm.at[pl.ds(base + c * K, K)], wsems[b])

        def wait_write(c, b):
            pltpu.make_async_copy(
                bufs[b], out_hbm.at[pl.ds(base + c * K, K)], wsems[b]).wait()

        for c in range(G):
            gather(c, c % NBUF)

        def step(c, j):
            wait_gather(c, j % NBUF)
            write(c, j % NBUF)

            @pl.when(c >= W)
            def _():
                wait_write(c - W, (j - W) % NBUF)

            @pl.when(c + G < N_CHUNKS)
            def _():
                gather(c + G, (j + G) % NBUF)

        def ring(i, carry):
            for j in range(NBUF):
                step(NBUF * i + j, j)
            return carry

        n_rounds = N_CHUNKS // NBUF
        lax.fori_loop(0, n_rounds, ring, 0)
        for c in range(n_rounds * NBUF, N_CHUNKS):
            step(jnp.int32(c), c % NBUF)
        for c in range(N_CHUNKS - W, N_CHUNKS):
            wait_write(jnp.int32(c), c % NBUF)

    return body(idx3, table)


def kernel(token_positions, PE):
    idx3 = token_positions.reshape(NW, N_CHUNKS, K)
    out = _pe_gather(idx3, PE)
    return out.reshape(token_positions.shape + (D_MODEL,))
